# baseline (speedup 1.0000x reference)
import functools

import jax
import jax.numpy as jnp
from jax import lax
from jax.experimental import pallas as pl
from jax.experimental.pallas import tpu as pltpu
from jax.experimental.pallas import tpu_sc as plsc

B, V, D, P = 1024, 100000, 16, 20

_NC, _NS = 2, 16
_NW = _NC * _NS
_BP = B * P
_BPW = _BP // _NW


def _sc_gather(table, idx):
    mesh = plsc.VectorSubcoreMesh(core_axis_name="c", subcore_axis_name="s")

    @functools.partial(
        pl.kernel,
        out_type=jax.ShapeDtypeStruct((_BP, D), jnp.float32),
        mesh=mesh,
        scratch_types=[
            pltpu.VMEM((_BPW,), jnp.int32),
            pltpu.VMEM((_BPW, D), jnp.float32),
            pltpu.SemaphoreType.DMA,
        ],
        compiler_params=pltpu.CompilerParams(use_tc_tiling_on_sc=False),
    )
    def k(table_hbm, idx_hbm, out_hbm, idx_v, rows_v, sem):
        wid = lax.axis_index("s") * _NC + lax.axis_index("c")
        base = wid * _BPW
        pltpu.sync_copy(idx_hbm.at[pl.ds(base, _BPW)], idx_v)
        pltpu.async_copy(table_hbm.at[idx_v], rows_v, sem).wait()
        pltpu.sync_copy(rows_v, out_hbm.at[pl.ds(base, _BPW)])

    return k(table, idx)


_MT = 32
_NBLK = B // _MT


_NS_TC = 4
_ST = _MT // _NS_TC


def _tc_body(iv0, iv1, iv2, iv3, w_ref, pvt_ref, hc_ref, out_ref, acc_ref):
    i = pl.program_id(0)

    @pl.when(i == 0)
    def _init():
        acc_ref[0] = 0.0

    w = w_ref[...].astype(jnp.bfloat16)
    t = hc_ref[...].astype(jnp.float32)
    bce_sum = 0.0
    for j, iv in enumerate((iv0, iv1, iv2, iv3)):
        x = lax.dot_general(iv[...].astype(jnp.bfloat16), w,
                            (((1,), (1,)), ((), ())),
                            preferred_element_type=jnp.float32)
        logits = jnp.zeros((_ST, P), jnp.float32)
        for d in range(D):
            logits = logits + pvt_ref[d, j * _ST:(j + 1) * _ST] * x[:, d:d + 1]
        tj = t[j * _ST:(j + 1) * _ST]
        bce = (jnp.maximum(logits, 0.0) - logits * tj
               + jnp.log1p(jnp.exp(-jnp.abs(logits))))
        bce_sum = bce_sum + jnp.sum(bce)
    acc_ref[0] += bce_sum

    @pl.when(i == _NBLK - 1)
    def _fin():
        out_ref[0, 0] = acc_ref[0] * (1.0 / (B * P))


def _stripe_spec(j):
    return pl.BlockSpec((_ST, V), lambda i, j=j: (_NS_TC * i + j, 0))


def _tc_loss(inputs_vector, W, pvt, hc):
    out = pl.pallas_call(
        _tc_body,
        grid=(_NBLK,),
        in_specs=[
            _stripe_spec(0), _stripe_spec(1), _stripe_spec(2), _stripe_spec(3),
            pl.BlockSpec((D, V), lambda i: (0, 0)),
            pl.BlockSpec((D, _MT, P), lambda i: (0, i, 0)),
            pl.BlockSpec((_MT, P), lambda i: (i, 0)),
        ],
        out_specs=pl.BlockSpec(memory_space=pltpu.SMEM),
        out_shape=jax.ShapeDtypeStruct((1, 1), jnp.float32),
        scratch_shapes=[pltpu.SMEM((1,), jnp.float32)],
        compiler_params=pltpu.CompilerParams(
            dimension_semantics=("arbitrary",),
            vmem_limit_bytes=100 * 1024 * 1024,
        ),
    )(inputs_vector, inputs_vector, inputs_vector, inputs_vector,
      W, pvt, hc)
    return out


def kernel(inputs_vector, path_nodes_indices, huffman_codes, W, cls):
    idx = path_nodes_indices.astype(jnp.int32).reshape(_BP)
    rows = _sc_gather(cls, idx)
    pvt = rows.reshape(B, P, D).transpose(2, 0, 1)
    loss = _tc_loss(inputs_vector, W, pvt,
                    huffman_codes.astype(jnp.int32))
    return loss.reshape(1)


# speedup vs baseline: 1.0973x; 1.0973x over previous
import functools

import jax
import jax.numpy as jnp
from jax import lax
from jax.experimental import pallas as pl
from jax.experimental.pallas import tpu as pltpu
from jax.experimental.pallas import tpu_sc as plsc

B, V, D, P = 1024, 100000, 16, 20

_NC, _NS = 2, 16
_NW = _NC * _NS
_BP = B * P
_BPW = _BP // _NW


def _sc_gather(table, idx):
    mesh = plsc.VectorSubcoreMesh(core_axis_name="c", subcore_axis_name="s")

    @functools.partial(
        pl.kernel,
        out_type=jax.ShapeDtypeStruct((_BP, D), jnp.float32),
        mesh=mesh,
        scratch_types=[
            pltpu.VMEM((_BPW,), jnp.int32),
            pltpu.VMEM((_BPW, D), jnp.float32),
            pltpu.SemaphoreType.DMA,
        ],
        compiler_params=pltpu.CompilerParams(use_tc_tiling_on_sc=False),
    )
    def k(table_hbm, idx_hbm, out_hbm, idx_v, rows_v, sem):
        wid = lax.axis_index("s") * _NC + lax.axis_index("c")
        base = wid * _BPW
        pltpu.sync_copy(idx_hbm.at[pl.ds(base, _BPW)], idx_v)
        pltpu.async_copy(table_hbm.at[idx_v], rows_v, sem).wait()
        pltpu.sync_copy(rows_v, out_hbm.at[pl.ds(base, _BPW)])

    return k(table, idx)


_MT = 32
_NBLK = B // _MT


_QS = 4
_CH = _MT // _QS


def _tc_body(hbm_ref, w_ref, pvt_ref, hc_ref, out_ref,
             bufa, bufb, acc_ref, sems):
    i = pl.program_id(0)
    slot = lax.rem(i, 2)

    def issue(step, buf, sl):
        for q in range(_QS):
            pltpu.make_async_copy(
                hbm_ref.at[pl.ds(step * _MT + q * _CH, _CH)],
                buf.at[pl.ds(q * _CH, _CH)],
                sems.at[sl, q],
            ).start()

    def drain(step, buf, sl):
        for q in range(_QS):
            pltpu.make_async_copy(
                hbm_ref.at[pl.ds(step * _MT + q * _CH, _CH)],
                buf.at[pl.ds(q * _CH, _CH)],
                sems.at[sl, q],
            ).wait()

    @pl.when(i == 0)
    def _prime():
        acc_ref[0] = 0.0
        issue(0, bufa, 0)

    @pl.when(i + 1 < _NBLK)
    def _next():
        @pl.when(slot == 0)
        def _():
            issue(i + 1, bufb, 1)

        @pl.when(slot == 1)
        def _():
            issue(i + 1, bufa, 0)

    def compute(buf):
        x = lax.dot_general(buf[...].astype(jnp.bfloat16),
                            w_ref[...].astype(jnp.bfloat16),
                            (((1,), (1,)), ((), ())),
                            preferred_element_type=jnp.float32)
        logits = jnp.zeros((_MT, P), jnp.float32)
        for d in range(D):
            logits = logits + pvt_ref[d] * x[:, d:d + 1]
        t = hc_ref[...].astype(jnp.float32)
        bce = (jnp.maximum(logits, 0.0) - logits * t
               + jnp.log1p(jnp.exp(-jnp.abs(logits))))
        acc_ref[0] += jnp.sum(bce)

    @pl.when(slot == 0)
    def _even():
        drain(i, bufa, 0)
        compute(bufa)

    @pl.when(slot == 1)
    def _odd():
        drain(i, bufb, 1)
        compute(bufb)

    @pl.when(i == _NBLK - 1)
    def _fin():
        out_ref[0, 0] = acc_ref[0] * (1.0 / (B * P))


def _tc_loss(inputs_vector, W, pvt, hc):
    out = pl.pallas_call(
        _tc_body,
        grid=(_NBLK,),
        in_specs=[
            pl.BlockSpec(memory_space=pl.ANY),
            pl.BlockSpec((D, V), lambda i: (0, 0)),
            pl.BlockSpec((D, _MT, P), lambda i: (0, i, 0)),
            pl.BlockSpec((_MT, P), lambda i: (i, 0)),
        ],
        out_specs=pl.BlockSpec(memory_space=pltpu.SMEM),
        out_shape=jax.ShapeDtypeStruct((1, 1), jnp.float32),
        scratch_shapes=[
            pltpu.VMEM((_MT, V), jnp.float32),
            pltpu.VMEM((_MT, V), jnp.float32),
            pltpu.SMEM((1,), jnp.float32),
            pltpu.SemaphoreType.DMA((2, _QS)),
        ],
        compiler_params=pltpu.CompilerParams(
            dimension_semantics=("arbitrary",),
            vmem_limit_bytes=100 * 1024 * 1024,
        ),
    )(inputs_vector, W, pvt, hc)
    return out


def kernel(inputs_vector, path_nodes_indices, huffman_codes, W, cls):
    idx = path_nodes_indices.astype(jnp.int32).reshape(_BP)
    rows = _sc_gather(cls, idx)
    pvt = rows.reshape(B, P, D).transpose(2, 0, 1)
    loss = _tc_loss(inputs_vector, W, pvt,
                    huffman_codes.astype(jnp.int32))
    return loss.reshape(1)


# speedup vs baseline: 2.8747x; 2.6197x over previous
import functools

import jax
import jax.numpy as jnp
from jax import lax
from jax.experimental import pallas as pl
from jax.experimental.pallas import tpu as pltpu
from jax.experimental.pallas import tpu_sc as plsc

B, V, D, P = 1024, 100000, 16, 20

_NC, _NS = 2, 16
_NW = _NC * _NS
_BP = B * P
_BPW = _BP // _NW


def _sc_gather(table, idx):
    mesh = plsc.VectorSubcoreMesh(core_axis_name="c", subcore_axis_name="s")

    @functools.partial(
        pl.kernel,
        out_type=jax.ShapeDtypeStruct((_BP, D), jnp.float32),
        mesh=mesh,
        scratch_types=[
            pltpu.VMEM((_BPW,), jnp.int32),
            pltpu.VMEM((_BPW, D), jnp.float32),
            pltpu.SemaphoreType.DMA,
        ],
        compiler_params=pltpu.CompilerParams(use_tc_tiling_on_sc=False),
    )
    def k(table_hbm, idx_hbm, out_hbm, idx_v, rows_v, sem):
        wid = lax.axis_index("s") * _NC + lax.axis_index("c")
        base = wid * _BPW
        pltpu.sync_copy(idx_hbm.at[pl.ds(base, _BPW)], idx_v)
        pltpu.async_copy(table_hbm.at[idx_v], rows_v, sem).wait()
        pltpu.sync_copy(rows_v, out_hbm.at[pl.ds(base, _BPW)])

    return k(table, idx)


_VT = 2048
_NBLK = (V + _VT - 1) // _VT
_VLAST = V - (_NBLK - 1) * _VT


def _tc_body(ivt_ref, w_ref, pvt_ref, hct_ref, out_ref, acc_ref):
    i = pl.program_id(0)

    @pl.when(i == 0)
    def _init():
        acc_ref[...] = jnp.zeros_like(acc_ref)

    def contrib(wb, ab):
        return lax.dot_general(wb, ab, (((1,), (0,)), ((), ())),
                               preferred_element_type=jnp.float32)

    @pl.when(i < _NBLK - 1)
    def _full():
        acc_ref[...] += contrib(w_ref[...].astype(jnp.bfloat16),
                                ivt_ref[...].astype(jnp.bfloat16))

    @pl.when(i == _NBLK - 1)
    def _last():
        mv = lax.broadcasted_iota(jnp.int32, (_VT, 1), 0) < _VLAST
        ab = jnp.where(mv, ivt_ref[...], 0.0)
        mw = lax.broadcasted_iota(jnp.int32, (1, _VT), 1) < _VLAST
        wb = jnp.where(mw, w_ref[...], 0.0)
        acc_ref[...] += contrib(wb.astype(jnp.bfloat16),
                                ab.astype(jnp.bfloat16))

        xt = acc_ref[...]
        logits = jnp.zeros((P, B), jnp.float32)
        for d in range(D):
            logits = logits + pvt_ref[d] * xt[d:d + 1, :]
        t = hct_ref[...].astype(jnp.float32)
        bce = (jnp.maximum(logits, 0.0) - logits * t
               + jnp.log1p(jnp.exp(-jnp.abs(logits))))
        out_ref[0, 0] = jnp.sum(bce) * (1.0 / (B * P))


def _tc_loss(ivt, W, pvt, hct):
    out = pl.pallas_call(
        _tc_body,
        grid=(_NBLK,),
        in_specs=[
            pl.BlockSpec((_VT, B), lambda i: (i, 0)),
            pl.BlockSpec((D, _VT), lambda i: (0, i)),
            pl.BlockSpec((D, P, B), lambda i: (0, 0, 0)),
            pl.BlockSpec((P, B), lambda i: (0, 0)),
        ],
        out_specs=pl.BlockSpec(memory_space=pltpu.SMEM),
        out_shape=jax.ShapeDtypeStruct((1, 1), jnp.float32),
        scratch_shapes=[pltpu.VMEM((D, B), jnp.float32)],
        compiler_params=pltpu.CompilerParams(
            dimension_semantics=("arbitrary",),
            vmem_limit_bytes=100 * 1024 * 1024,
        ),
    )(ivt, W, pvt, hct)
    return out


def kernel(inputs_vector, path_nodes_indices, huffman_codes, W, cls):
    idx = path_nodes_indices.astype(jnp.int32).reshape(_BP)
    rows = _sc_gather(cls, idx)
    pvt = rows.reshape(B, P, D).transpose(2, 1, 0)
    ivt = inputs_vector.T
    hct = huffman_codes.astype(jnp.int32).T
    loss = _tc_loss(ivt, W, pvt, hct)
    return loss.reshape(1)


# speedup vs baseline: 3.1407x; 1.0925x over previous
import functools

import jax
import jax.numpy as jnp
from jax import lax
from jax.experimental import pallas as pl
from jax.experimental.pallas import tpu as pltpu
from jax.experimental.pallas import tpu_sc as plsc

B, V, D, P = 1024, 100000, 16, 20

_NC, _NS = 2, 16
_NW = _NC * _NS
_BP = B * P
_BPW = _BP // _NW


def _sc_gather(table, idx):
    mesh = plsc.VectorSubcoreMesh(core_axis_name="c", subcore_axis_name="s")

    @functools.partial(
        pl.kernel,
        out_type=jax.ShapeDtypeStruct((_BP, D), jnp.float32),
        mesh=mesh,
        scratch_types=[
            pltpu.VMEM((_BPW,), jnp.int32),
            pltpu.VMEM((_BPW, D), jnp.float32),
            pltpu.SemaphoreType.DMA,
        ],
        compiler_params=pltpu.CompilerParams(use_tc_tiling_on_sc=False),
    )
    def k(table_hbm, idx_hbm, out_hbm, idx_v, rows_v, sem):
        wid = lax.axis_index("s") * _NC + lax.axis_index("c")
        base = wid * _BPW
        pltpu.sync_copy(idx_hbm.at[pl.ds(base, _BPW)], idx_v)
        pltpu.async_copy(table_hbm.at[idx_v], rows_v, sem).wait()
        pltpu.sync_copy(rows_v, out_hbm.at[pl.ds(base, _BPW)])

    return k(table, idx)


_VT = 2048
_NBLK = (V + _VT - 1) // _VT
_VLAST = V - (_NBLK - 1) * _VT


def _mm_body(ivt_ref, w_ref, out_ref):
    i = pl.program_id(0)

    @pl.when(i == 0)
    def _init():
        out_ref[...] = jnp.zeros_like(out_ref)

    def contrib(wb, ab):
        return lax.dot_general(wb, ab, (((1,), (0,)), ((), ())),
                               preferred_element_type=jnp.float32)

    @pl.when(i < _NBLK - 1)
    def _full():
        out_ref[...] += contrib(w_ref[...].astype(jnp.bfloat16),
                                ivt_ref[...].astype(jnp.bfloat16))

    @pl.when(i == _NBLK - 1)
    def _last():
        mv = lax.broadcasted_iota(jnp.int32, (_VT, 1), 0) < _VLAST
        ab = jnp.where(mv, ivt_ref[...], 0.0)
        mw = lax.broadcasted_iota(jnp.int32, (1, _VT), 1) < _VLAST
        wb = jnp.where(mw, w_ref[...], 0.0)
        out_ref[...] += contrib(wb.astype(jnp.bfloat16),
                                ab.astype(jnp.bfloat16))


def _tc_matmul(ivt, W):
    return pl.pallas_call(
        _mm_body,
        grid=(_NBLK,),
        in_specs=[
            pl.BlockSpec((_VT, B), lambda i: (i, 0)),
            pl.BlockSpec((D, _VT), lambda i: (0, i)),
        ],
        out_specs=pl.BlockSpec((D, B), lambda i: (0, 0)),
        out_shape=jax.ShapeDtypeStruct((D, B), jnp.float32),
        compiler_params=pltpu.CompilerParams(
            dimension_semantics=("arbitrary",),
            vmem_limit_bytes=100 * 1024 * 1024,
        ),
    )(ivt, W)


def _ep_body(xt_ref, rows_ref, hct_ref, out_ref):
    pvt = jnp.transpose(rows_ref[...])
    resh = pvt.reshape(D * P, B)
    xt = xt_ref[...]
    logits = jnp.zeros((P, B), jnp.float32)
    for d in range(D):
        logits = logits + resh[d * P:(d + 1) * P] * xt[d:d + 1, :]
    t = hct_ref[...].astype(jnp.float32)
    bce = (jnp.maximum(logits, 0.0) - logits * t
           + jnp.log1p(jnp.exp(-jnp.abs(logits))))
    out_ref[0, 0] = jnp.sum(bce) * (1.0 / (B * P))


def _tc_epilogue(xt, rows, hct):
    return pl.pallas_call(
        _ep_body,
        out_specs=pl.BlockSpec(memory_space=pltpu.SMEM),
        out_shape=jax.ShapeDtypeStruct((1, 1), jnp.float32),
    )(xt, rows, hct)


def kernel(inputs_vector, path_nodes_indices, huffman_codes, W, cls):
    idx = path_nodes_indices.astype(jnp.int32).T.reshape(_BP)
    rows = _sc_gather(cls, idx)
    ivt = inputs_vector.T
    hct = huffman_codes.astype(jnp.int32).T
    xt = _tc_matmul(ivt, W)
    loss = _tc_epilogue(xt, rows, hct)
    return loss.reshape(1)


# speedup vs baseline: 3.9284x; 1.2508x over previous
import functools

import jax
import jax.numpy as jnp
from jax import lax
from jax.experimental import pallas as pl
from jax.experimental.pallas import tpu as pltpu
from jax.experimental.pallas import tpu_sc as plsc

B, V, D, P = 1024, 100000, 16, 20
_NROWS = V - 1

_NC, _NS = 2, 16
_NW = _NC * _NS
_BP = B * P
_NE = D * _BP
_EW = _NE // _NW


def _sc_gather(table1d, eidx):
    mesh = plsc.VectorSubcoreMesh(core_axis_name="c", subcore_axis_name="s")

    @functools.partial(
        pl.kernel,
        out_type=jax.ShapeDtypeStruct((_NE,), jnp.float32),
        mesh=mesh,
        scratch_types=[
            pltpu.VMEM((_EW,), jnp.int32),
            pltpu.VMEM((_EW,), jnp.float32),
            pltpu.SemaphoreType.DMA,
        ],
        compiler_params=pltpu.CompilerParams(use_tc_tiling_on_sc=False),
    )
    def k(tab_hbm, eidx_hbm, out_hbm, idx_v, vals_v, sem):
        wid = lax.axis_index("s") * _NC + lax.axis_index("c")
        base = wid * _EW
        pltpu.sync_copy(eidx_hbm.at[pl.ds(base, _EW)], idx_v)
        pltpu.async_copy(tab_hbm.at[idx_v], vals_v, sem).wait()
        pltpu.sync_copy(vals_v, out_hbm.at[pl.ds(base, _EW)])

    return k(table1d, eidx)


_VT = 2048
_NBLK = (V + _VT - 1) // _VT
_VLAST = V - (_NBLK - 1) * _VT


def _mm_body(ivt_ref, w_ref, out_ref):
    i = pl.program_id(0)

    @pl.when(i == 0)
    def _init():
        out_ref[...] = jnp.zeros_like(out_ref)

    def contrib(wb, ab):
        return lax.dot_general(wb, ab, (((1,), (0,)), ((), ())),
                               preferred_element_type=jnp.float32)

    @pl.when(i < _NBLK - 1)
    def _full():
        out_ref[...] += contrib(w_ref[...].astype(jnp.bfloat16),
                                ivt_ref[...].astype(jnp.bfloat16))

    @pl.when(i == _NBLK - 1)
    def _last():
        mv = lax.broadcasted_iota(jnp.int32, (_VT, 1), 0) < _VLAST
        ab = jnp.where(mv, ivt_ref[...], 0.0)
        mw = lax.broadcasted_iota(jnp.int32, (1, _VT), 1) < _VLAST
        wb = jnp.where(mw, w_ref[...], 0.0)
        out_ref[...] += contrib(wb.astype(jnp.bfloat16),
                                ab.astype(jnp.bfloat16))


def _tc_matmul(ivt, W):
    return pl.pallas_call(
        _mm_body,
        grid=(_NBLK,),
        in_specs=[
            pl.BlockSpec((_VT, B), lambda i: (i, 0)),
            pl.BlockSpec((D, _VT), lambda i: (0, i)),
        ],
        out_specs=pl.BlockSpec((D, B), lambda i: (0, 0)),
        out_shape=jax.ShapeDtypeStruct((D, B), jnp.float32),
        compiler_params=pltpu.CompilerParams(
            dimension_semantics=("arbitrary",),
            vmem_limit_bytes=100 * 1024 * 1024,
        ),
    )(ivt, W)


def _ep_body(xt_ref, pvt_ref, hct_ref, out_ref):
    xt = xt_ref[...]
    logits = jnp.zeros((P, B), jnp.float32)
    for d in range(D):
        logits = logits + pvt_ref[d] * xt[d:d + 1, :]
    t = hct_ref[...].astype(jnp.float32)
    bce = (jnp.maximum(logits, 0.0) - logits * t
           + jnp.log1p(jnp.exp(-jnp.abs(logits))))
    out_ref[0, 0] = jnp.sum(bce) * (1.0 / (B * P))


def _tc_epilogue(xt, pvt, hct):
    return pl.pallas_call(
        _ep_body,
        out_specs=pl.BlockSpec(memory_space=pltpu.SMEM),
        out_shape=jax.ShapeDtypeStruct((1, 1), jnp.float32),
    )(xt, pvt, hct)


def kernel(inputs_vector, path_nodes_indices, huffman_codes, W, cls):
    idx = path_nodes_indices.astype(jnp.int32).T.reshape(_BP)
    eidx = (jnp.arange(D, dtype=jnp.int32)[:, None] * _NROWS
            + idx[None, :]).reshape(_NE)
    table1d = cls.T.reshape(_NROWS * D)
    vals = _sc_gather(table1d, eidx)
    pvt = vals.reshape(D, P, B)
    ivt = inputs_vector.T
    hct = huffman_codes.astype(jnp.int32).T
    xt = _tc_matmul(ivt, W)
    loss = _tc_epilogue(xt, pvt, hct)
    return loss.reshape(1)


# speedup vs baseline: 3.9296x; 1.0003x over previous
import functools

import jax
import jax.numpy as jnp
from jax import lax
from jax.experimental import pallas as pl
from jax.experimental.pallas import tpu as pltpu
from jax.experimental.pallas import tpu_sc as plsc

B, V, D, P = 1024, 100000, 16, 20
_NROWS = V - 1

_NC, _NS = 2, 16
_NW = _NC * _NS
_BP = B * P
_NE = D * _BP
_EW = _NE // _NW


def _sc_gather(table1d, eidx):
    mesh = plsc.VectorSubcoreMesh(core_axis_name="c", subcore_axis_name="s")

    @functools.partial(
        pl.kernel,
        out_type=jax.ShapeDtypeStruct((_NE,), jnp.float32),
        mesh=mesh,
        scratch_types=[
            pltpu.VMEM((_EW,), jnp.int32),
            pltpu.VMEM((_EW,), jnp.float32),
            pltpu.SemaphoreType.DMA,
        ],
        compiler_params=pltpu.CompilerParams(use_tc_tiling_on_sc=False),
    )
    def k(tab_hbm, eidx_hbm, out_hbm, idx_v, vals_v, sem):
        wid = lax.axis_index("s") * _NC + lax.axis_index("c")
        base = wid * _EW
        pltpu.sync_copy(eidx_hbm.at[pl.ds(base, _EW)], idx_v)
        pltpu.async_copy(tab_hbm.at[idx_v], vals_v, sem).wait()
        pltpu.sync_copy(vals_v, out_hbm.at[pl.ds(base, _EW)])

    return k(table1d, eidx)


_VT = 4096
_NBLK = (V + _VT - 1) // _VT
_VLAST = V - (_NBLK - 1) * _VT


def _mm_body(ivt_ref, w_ref, out_ref):
    i = pl.program_id(0)

    @pl.when(i == 0)
    def _init():
        out_ref[...] = jnp.zeros_like(out_ref)

    def contrib(wb, ab):
        return lax.dot_general(wb, ab, (((1,), (0,)), ((), ())),
                               preferred_element_type=jnp.float32)

    @pl.when(i < _NBLK - 1)
    def _full():
        out_ref[...] += contrib(w_ref[...].astype(jnp.bfloat16),
                                ivt_ref[...].astype(jnp.bfloat16))

    @pl.when(i == _NBLK - 1)
    def _last():
        mv = lax.broadcasted_iota(jnp.int32, (_VT, 1), 0) < _VLAST
        ab = jnp.where(mv, ivt_ref[...], 0.0)
        mw = lax.broadcasted_iota(jnp.int32, (1, _VT), 1) < _VLAST
        wb = jnp.where(mw, w_ref[...], 0.0)
        out_ref[...] += contrib(wb.astype(jnp.bfloat16),
                                ab.astype(jnp.bfloat16))


def _tc_matmul(ivt, W):
    return pl.pallas_call(
        _mm_body,
        grid=(_NBLK,),
        in_specs=[
            pl.BlockSpec((_VT, B), lambda i: (i, 0)),
            pl.BlockSpec((D, _VT), lambda i: (0, i)),
        ],
        out_specs=pl.BlockSpec((D, B), lambda i: (0, 0)),
        out_shape=jax.ShapeDtypeStruct((D, B), jnp.float32),
        compiler_params=pltpu.CompilerParams(
            dimension_semantics=("arbitrary",),
            vmem_limit_bytes=100 * 1024 * 1024,
        ),
    )(ivt, W)


def _ep_body(xt_ref, pvt_ref, hct_ref, out_ref):
    xt = xt_ref[...]
    logits = jnp.zeros((P, B), jnp.float32)
    for d in range(D):
        logits = logits + pvt_ref[d] * xt[d:d + 1, :]
    t = hct_ref[...].astype(jnp.float32)
    bce = (jnp.maximum(logits, 0.0) - logits * t
           + jnp.log1p(jnp.exp(-jnp.abs(logits))))
    out_ref[0, 0] = jnp.sum(bce) * (1.0 / (B * P))


def _tc_epilogue(xt, pvt, hct):
    return pl.pallas_call(
        _ep_body,
        out_specs=pl.BlockSpec(memory_space=pltpu.SMEM),
        out_shape=jax.ShapeDtypeStruct((1, 1), jnp.float32),
    )(xt, pvt, hct)


def kernel(inputs_vector, path_nodes_indices, huffman_codes, W, cls):
    idx = path_nodes_indices.astype(jnp.int32).T.reshape(_BP)
    eidx = (jnp.arange(D, dtype=jnp.int32)[:, None] * _NROWS
            + idx[None, :]).reshape(_NE)
    table1d = cls.T.reshape(_NROWS * D)
    vals = _sc_gather(table1d, eidx)
    pvt = vals.reshape(D, P, B)
    ivt = inputs_vector.T
    hct = huffman_codes.astype(jnp.int32).T
    xt = _tc_matmul(ivt, W)
    loss = _tc_epilogue(xt, pvt, hct)
    return loss.reshape(1)
